# baseline (speedup 1.0000x reference)
import functools

import jax
import jax.numpy as jnp
from jax import lax
from jax.experimental import pallas as pl
from jax.experimental.pallas import tpu as pltpu
from jax.experimental.pallas import tpu_sc as plsc

K = 32
L = 16
NC = 2
NS = 16
NW = NC * NS
B = 16384
BPW = B // NW
CH = 128
NCH = BPW // CH


def _permute(x, idx):
    dnums = lax.GatherDimensionNumbers(
        offset_dims=(), collapsed_slice_dims=(0,), start_index_map=(0,))
    return lax.gather(x, idx[:, None], dnums, (1,),
                      mode=lax.GatherScatterMode.PROMISE_IN_BOUNDS)


def _mf_body(o_hbm, m_hbm, w_hbm, u_hbm, mt_hbm, e_hbm, out_hbm,
             o_idx, m_idx, u_rows, m_rows, e_rows, w_v, out_v, sem):
    wid = lax.axis_index("s") * NC + lax.axis_index("c")
    base = wid * BPW

    for i in range(NCH):
        pltpu.sync_copy(o_hbm.at[pl.ds(base + i * CH, CH)], o_idx.at[i])
        pltpu.sync_copy(m_hbm.at[pl.ds(base + i * CH, CH)], m_idx.at[i])
    pltpu.sync_copy(w_hbm.at[pl.ds(base, BPW)], w_v)

    copies = []
    for i in range(NCH):
        sl = pl.ds(i * CH, CH)
        copies.append(pltpu.async_copy(u_hbm.at[o_idx.at[i]], u_rows.at[sl], sem))
        copies.append(pltpu.async_copy(e_hbm.at[o_idx.at[i]], e_rows.at[sl], sem))
        copies.append(pltpu.async_copy(mt_hbm.at[m_idx.at[i]], m_rows.at[sl], sem))
    for c in copies:
        c.wait()

    lanes = lax.iota(jnp.int32, L)

    def group(g, carry):
        acc_u = jnp.zeros((L,), jnp.float32)
        acc_e = jnp.zeros((L,), jnp.float32)
        for j in range(L):
            b = g * L + j
            mva = m_rows[b, pl.ds(0, L)]
            mvb = m_rows[b, pl.ds(L, L)]
            ua = u_rows[b, pl.ds(0, L)]
            ub = u_rows[b, pl.ds(L, L)]
            ea = e_rows[b, pl.ds(0, L)]
            eb = e_rows[b, pl.ds(L, L)]
            pu = mva * ua + mvb * ub
            pe = mva * ea + mvb * eb
            for sh in (8, 4, 2, 1):
                perm = jnp.bitwise_xor(lanes, sh)
                pu = pu + _permute(pu, perm)
                pe = pe + _permute(pe, perm)
            acc_u = jnp.where(lanes == j, pu, acc_u)
            acc_e = jnp.where(lanes == j, pe, acc_e)
        wv = w_v[pl.ds(g * L, L)]
        out_v[pl.ds(g * L, L)] = acc_u * wv + acc_e * (1.0 - wv)
        return carry

    lax.fori_loop(0, BPW // L, group, 0)
    pltpu.sync_copy(out_v, out_hbm.at[pl.ds(base, BPW)])


def kernel(o, m, is_user, U, M, E):
    w = is_user.reshape(-1).astype(jnp.float32)
    o32 = o.astype(jnp.int32)
    m32 = m.astype(jnp.int32)
    mesh = plsc.VectorSubcoreMesh(core_axis_name="c", subcore_axis_name="s")
    run = pl.kernel(
        _mf_body,
        mesh=mesh,
        compiler_params=pltpu.CompilerParams(use_tc_tiling_on_sc=False),
        out_type=jax.ShapeDtypeStruct((B,), jnp.float32),
        scratch_types=[
            pltpu.VMEM((NCH, CH), jnp.int32),
            pltpu.VMEM((NCH, CH), jnp.int32),
            pltpu.VMEM((BPW, K), jnp.float32),
            pltpu.VMEM((BPW, K), jnp.float32),
            pltpu.VMEM((BPW, K), jnp.float32),
            pltpu.VMEM((BPW,), jnp.float32),
            pltpu.VMEM((BPW,), jnp.float32),
            pltpu.SemaphoreType.DMA,
        ],
    )
    return run(o32, m32, w, U, M, E)
